# initial kernel scaffold (unmeasured)
import jax
import jax.numpy as jnp
from jax import lax
from jax.experimental import pallas as pl
from jax.experimental.pallas import tpu as pltpu

N_DEV = 4
SQ = 2048
D_MODEL = 1024
HQ = 8
DH = 128
SCALE = 0.08838834764831843
BLK = 64
N_BLK = SQ // BLK
GROUPS = 4
BLK_PER_GROUP = N_BLK // GROUPS
CHUNK = SQ // N_DEV


def _body(x_ref, wq_ref, k_ref, v_ref, wo_ref, out_ref,
          q_ref, ctx_ref, comm_ref, send_sems, recv_sems):
    my = lax.axis_index("i")
    right = lax.rem(my + 1, N_DEV)
    left = lax.rem(my + N_DEV - 1, N_DEV)

    for c in range(N_DEV):
        rows = slice(CHUNK * c, CHUNK * (c + 1))
        q_ref[rows, :] = jnp.dot(
            x_ref[rows, :], wq_ref[:, :], preferred_element_type=jnp.float32
        )

    for r in range(GROUPS):
        offs = [BLK * (GROUPS * m + r) for m in range(BLK_PER_GROUP)]
        Kr = jnp.concatenate([k_ref[o:o + BLK, :] for o in offs], axis=0)
        Vr = jnp.concatenate([v_ref[o:o + BLK, :] for o in offs], axis=0)
        Qr = jnp.concatenate([q_ref[o:o + BLK, :] for o in offs], axis=0)
        for h in range(HQ):
            cs = slice(DH * h, DH * (h + 1))
            s = lax.dot_general(
                Qr[:, cs], Kr[:, cs],
                dimension_numbers=(((1,), (1,)), ((), ())),
                preferred_element_type=jnp.float32,
            ) * SCALE
            s = s - jnp.max(s, axis=-1, keepdims=True)
            e = jnp.exp(s)
            w = e / jnp.sum(e, axis=-1, keepdims=True)
            ctx_h = jnp.dot(w, Vr[:, cs], preferred_element_type=jnp.float32)
            for m in range(BLK_PER_GROUP):
                ctx_ref[offs[m]:offs[m] + BLK, cs] = ctx_h[BLK * m:BLK * (m + 1), :]

    for c in range(N_DEV):
        rows = slice(CHUNK * c, CHUNK * (c + 1))
        out_ref[rows, :] = jnp.dot(
            ctx_ref[rows, :], wo_ref[:, :], preferred_element_type=jnp.float32
        )

    barrier_sem = pltpu.get_barrier_semaphore()
    for nbr in (left, right):
        pl.semaphore_signal(
            barrier_sem, inc=1,
            device_id=(nbr,), device_id_type=pl.DeviceIdType.MESH,
        )
    pl.semaphore_wait(barrier_sem, 2)

    for h in range(N_DEV - 1):
        slot = h % 2
        c_s = lax.rem(my + N_DEV - h, N_DEV)
        c_r = lax.rem(my + 2 * N_DEV - h - 1, N_DEV)
        rdma = pltpu.make_async_remote_copy(
            src_ref=out_ref.at[pl.ds(c_s * CHUNK, CHUNK), :],
            dst_ref=comm_ref.at[slot],
            send_sem=send_sems.at[slot],
            recv_sem=recv_sems.at[slot],
            device_id=(right,),
            device_id_type=pl.DeviceIdType.MESH,
        )
        rdma.start()
        rdma.wait()
        dst = pl.ds(c_r * CHUNK, CHUNK)
        out_ref[dst, :] = out_ref[dst, :] + comm_ref[slot, :, :]

    for g in range(N_DEV - 1):
        slot = (N_DEV - 1 + g) % 2
        c_s = lax.rem(my + 1 + N_DEV - g, N_DEV)
        src = pl.ds(c_s * CHUNK, CHUNK)
        rdma = pltpu.make_async_remote_copy(
            src_ref=out_ref.at[src, :],
            dst_ref=out_ref.at[src, :],
            send_sem=send_sems.at[slot],
            recv_sem=recv_sems.at[slot],
            device_id=(right,),
            device_id_type=pl.DeviceIdType.MESH,
        )
        rdma.start()
        rdma.wait()


def kernel(x, Wq, K_ext, V_ext, Wo):
    i = lax.axis_index("i")
    x2 = x.reshape(SQ, D_MODEL)
    K = lax.dynamic_slice_in_dim(
        K_ext.reshape(SQ, 32, DH), i * HQ, HQ, axis=1
    ).reshape(SQ, HQ * DH)
    V = lax.dynamic_slice_in_dim(
        V_ext.reshape(SQ, 32, DH), i * HQ, HQ, axis=1
    ).reshape(SQ, HQ * DH)

    out = pl.pallas_call(
        _body,
        out_shape=jax.ShapeDtypeStruct((SQ, D_MODEL), jnp.float32),
        in_specs=[pl.BlockSpec(memory_space=pltpu.VMEM)] * 5,
        out_specs=pl.BlockSpec(memory_space=pltpu.VMEM),
        scratch_shapes=[
            pltpu.VMEM((SQ, HQ * DH), jnp.float32),
            pltpu.VMEM((SQ, HQ * DH), jnp.float32),
            pltpu.VMEM((2, CHUNK, D_MODEL), jnp.float32),
            pltpu.SemaphoreType.DMA((2,)),
            pltpu.SemaphoreType.DMA((2,)),
        ],
        compiler_params=pltpu.CompilerParams(collective_id=0),
    )(x2, Wq, K, V, Wo)
    return out.reshape(1, SQ, D_MODEL)


# baseline (device time: 211651 ns/iter reference)
import jax
import jax.numpy as jnp
from jax import lax
from jax.experimental import pallas as pl
from jax.experimental.pallas import tpu as pltpu

N_DEV = 4
SQ = 2048
D_MODEL = 1024
HQ = 8
DH = 128
SCALE = 0.08838834764831843
BLK = 64
N_BLK = SQ // BLK
GROUPS = 4
BLK_PER_GROUP = N_BLK // GROUPS
CHUNK = SQ // N_DEV


def _body(x_ref, wq_ref, k_ref, v_ref, wo_ref, out_ref,
          comm_ref, send_sems, recv_sems):
    my = lax.axis_index("i")
    right = lax.rem(my + 1, N_DEV)
    left = lax.rem(my + N_DEV - 1, N_DEV)

    for r in range(GROUPS):
        offs = [BLK * (GROUPS * m + r) for m in range(BLK_PER_GROUP)]
        Xr = jnp.concatenate([x_ref[o:o + BLK, :] for o in offs], axis=0)
        Kr = jnp.concatenate([k_ref[o:o + BLK, :] for o in offs], axis=0)
        Vr = jnp.concatenate([v_ref[o:o + BLK, :] for o in offs], axis=0)
        Qr = jnp.dot(Xr, wq_ref[:, :], preferred_element_type=jnp.float32)
        ctx_parts = []
        for h in range(HQ):
            cs = slice(DH * h, DH * (h + 1))
            s = lax.dot_general(
                Qr[:, cs], Kr[:, cs],
                dimension_numbers=(((1,), (1,)), ((), ())),
                preferred_element_type=jnp.float32,
            ) * SCALE
            s = s - jnp.max(s, axis=-1, keepdims=True)
            e = jnp.exp(s)
            w = e / jnp.sum(e, axis=-1, keepdims=True)
            ctx_parts.append(
                jnp.dot(w, Vr[:, cs], preferred_element_type=jnp.float32)
            )
        ctx_r = jnp.concatenate(ctx_parts, axis=1)
        out_r = jnp.dot(ctx_r, wo_ref[:, :], preferred_element_type=jnp.float32)
        for m in range(BLK_PER_GROUP):
            out_ref[offs[m]:offs[m] + BLK, :] = out_r[BLK * m:BLK * (m + 1), :]

    barrier_sem = pltpu.get_barrier_semaphore()
    for nbr in (left, right):
        pl.semaphore_signal(
            barrier_sem, inc=1,
            device_id=(nbr,), device_id_type=pl.DeviceIdType.MESH,
        )
    pl.semaphore_wait(barrier_sem, 2)

    for h in range(N_DEV - 1):
        slot = h % 2
        c_s = lax.rem(my + N_DEV - h, N_DEV)
        c_r = lax.rem(my + 2 * N_DEV - h - 1, N_DEV)
        rdma = pltpu.make_async_remote_copy(
            src_ref=out_ref.at[pl.ds(c_s * CHUNK, CHUNK), :],
            dst_ref=comm_ref.at[slot],
            send_sem=send_sems.at[slot],
            recv_sem=recv_sems.at[slot],
            device_id=(right,),
            device_id_type=pl.DeviceIdType.MESH,
        )
        rdma.start()
        rdma.wait()
        dst = pl.ds(c_r * CHUNK, CHUNK)
        out_ref[dst, :] = out_ref[dst, :] + comm_ref[slot, :, :]

    for g in range(N_DEV - 1):
        slot = (N_DEV - 1 + g) % 2
        c_s = lax.rem(my + 1 + N_DEV - g, N_DEV)
        src = pl.ds(c_s * CHUNK, CHUNK)
        rdma = pltpu.make_async_remote_copy(
            src_ref=out_ref.at[src, :],
            dst_ref=out_ref.at[src, :],
            send_sem=send_sems.at[slot],
            recv_sem=recv_sems.at[slot],
            device_id=(right,),
            device_id_type=pl.DeviceIdType.MESH,
        )
        rdma.start()
        rdma.wait()


def kernel(x, Wq, K_ext, V_ext, Wo):
    i = lax.axis_index("i")
    x2 = x.reshape(SQ, D_MODEL)
    K = lax.dynamic_slice_in_dim(
        K_ext.reshape(SQ, 32, DH), i * HQ, HQ, axis=1
    ).reshape(SQ, HQ * DH)
    V = lax.dynamic_slice_in_dim(
        V_ext.reshape(SQ, 32, DH), i * HQ, HQ, axis=1
    ).reshape(SQ, HQ * DH)

    out = pl.pallas_call(
        _body,
        out_shape=jax.ShapeDtypeStruct((SQ, D_MODEL), jnp.float32),
        in_specs=[pl.BlockSpec(memory_space=pltpu.VMEM)] * 5,
        out_specs=pl.BlockSpec(memory_space=pltpu.VMEM),
        scratch_shapes=[
            pltpu.VMEM((2, CHUNK, D_MODEL), jnp.float32),
            pltpu.SemaphoreType.DMA((2,)),
            pltpu.SemaphoreType.DMA((2,)),
        ],
        compiler_params=pltpu.CompilerParams(
            collective_id=0,
            vmem_limit_bytes=100 * 1024 * 1024,
        ),
    )(x2, Wq, K, V, Wo)
    return out.reshape(1, SQ, D_MODEL)


# device time: 144038 ns/iter; 1.4694x vs baseline; 1.4694x over previous
import jax
import jax.numpy as jnp
from jax import lax
from jax.experimental import pallas as pl
from jax.experimental.pallas import tpu as pltpu

N_DEV = 4
SQ = 2048
D_MODEL = 1024
HQ = 8
DH = 128
SCALE = 0.08838834764831843
BLK = 64
N_BLK = SQ // BLK
GROUPS = 4
BLK_PER_GROUP = N_BLK // GROUPS
CHUNK = SQ // N_DEV


HALFC = CHUNK // 2


def _body(x_ref, wq_ref, k_ref, v_ref, wo_ref, out_ref,
          comm_r, comm_l, send_r, recv_r, send_l, recv_l):
    my = lax.axis_index("i")
    right = lax.rem(my + 1, N_DEV)
    left = lax.rem(my + N_DEV - 1, N_DEV)

    for r in range(GROUPS):
        offs = [BLK * (GROUPS * m + r) for m in range(BLK_PER_GROUP)]
        Xr = jnp.concatenate([x_ref[o:o + BLK, :] for o in offs], axis=0)
        Kr = jnp.concatenate([k_ref[o:o + BLK, :] for o in offs], axis=0)
        Vr = jnp.concatenate([v_ref[o:o + BLK, :] for o in offs], axis=0)
        Qr = jnp.dot(Xr, wq_ref[:, :], preferred_element_type=jnp.float32)
        ctx_parts = []
        for h in range(HQ):
            cs = slice(DH * h, DH * (h + 1))
            s = lax.dot_general(
                Qr[:, cs], Kr[:, cs],
                dimension_numbers=(((1,), (1,)), ((), ())),
                preferred_element_type=jnp.float32,
            ) * SCALE
            s = s - jnp.max(s, axis=-1, keepdims=True)
            e = jnp.exp(s)
            w = e / jnp.sum(e, axis=-1, keepdims=True)
            ctx_parts.append(
                jnp.dot(w, Vr[:, cs], preferred_element_type=jnp.float32)
            )
        ctx_r = jnp.concatenate(ctx_parts, axis=1)
        out_r = jnp.dot(ctx_r, wo_ref[:, :], preferred_element_type=jnp.float32)
        for m in range(BLK_PER_GROUP):
            out_ref[offs[m]:offs[m] + BLK, :] = out_r[BLK * m:BLK * (m + 1), :]

    barrier_sem = pltpu.get_barrier_semaphore()
    for nbr in (left, right):
        pl.semaphore_signal(
            barrier_sem, inc=1,
            device_id=(nbr,), device_id_type=pl.DeviceIdType.MESH,
        )
    pl.semaphore_wait(barrier_sem, 2)

    for h in range(N_DEV - 1):
        slot = h % 2
        c_sr = lax.rem(my + N_DEV - h, N_DEV)
        c_rr = lax.rem(my + 2 * N_DEV - h - 1, N_DEV)
        c_sl = lax.rem(my + h, N_DEV)
        c_rl = lax.rem(my + h + 1, N_DEV)
        rdma_r = pltpu.make_async_remote_copy(
            src_ref=out_ref.at[pl.ds(c_sr * CHUNK, HALFC), :],
            dst_ref=comm_r.at[slot],
            send_sem=send_r.at[slot],
            recv_sem=recv_r.at[slot],
            device_id=(right,),
            device_id_type=pl.DeviceIdType.MESH,
        )
        rdma_l = pltpu.make_async_remote_copy(
            src_ref=out_ref.at[pl.ds(c_sl * CHUNK + HALFC, HALFC), :],
            dst_ref=comm_l.at[slot],
            send_sem=send_l.at[slot],
            recv_sem=recv_l.at[slot],
            device_id=(left,),
            device_id_type=pl.DeviceIdType.MESH,
        )
        rdma_r.start()
        rdma_l.start()
        rdma_r.wait()
        dst = pl.ds(c_rr * CHUNK, HALFC)
        out_ref[dst, :] = out_ref[dst, :] + comm_r[slot, :, :]
        rdma_l.wait()
        dst = pl.ds(c_rl * CHUNK + HALFC, HALFC)
        out_ref[dst, :] = out_ref[dst, :] + comm_l[slot, :, :]

    for g in range(N_DEV - 1):
        slot = (N_DEV - 1 + g) % 2
        c_sr = lax.rem(my + 1 + N_DEV - g, N_DEV)
        c_sl = lax.rem(my + N_DEV - 1 + g, N_DEV)
        src_r = pl.ds(c_sr * CHUNK, HALFC)
        src_l = pl.ds(c_sl * CHUNK + HALFC, HALFC)
        rdma_r = pltpu.make_async_remote_copy(
            src_ref=out_ref.at[src_r, :],
            dst_ref=out_ref.at[src_r, :],
            send_sem=send_r.at[slot],
            recv_sem=recv_r.at[slot],
            device_id=(right,),
            device_id_type=pl.DeviceIdType.MESH,
        )
        rdma_l = pltpu.make_async_remote_copy(
            src_ref=out_ref.at[src_l, :],
            dst_ref=out_ref.at[src_l, :],
            send_sem=send_l.at[slot],
            recv_sem=recv_l.at[slot],
            device_id=(left,),
            device_id_type=pl.DeviceIdType.MESH,
        )
        rdma_r.start()
        rdma_l.start()
        rdma_r.wait()
        rdma_l.wait()


def kernel(x, Wq, K_ext, V_ext, Wo):
    i = lax.axis_index("i")
    x2 = x.reshape(SQ, D_MODEL)
    K = lax.dynamic_slice_in_dim(
        K_ext.reshape(SQ, 32, DH), i * HQ, HQ, axis=1
    ).reshape(SQ, HQ * DH)
    V = lax.dynamic_slice_in_dim(
        V_ext.reshape(SQ, 32, DH), i * HQ, HQ, axis=1
    ).reshape(SQ, HQ * DH)

    out = pl.pallas_call(
        _body,
        out_shape=jax.ShapeDtypeStruct((SQ, D_MODEL), jnp.float32),
        in_specs=[pl.BlockSpec(memory_space=pltpu.VMEM)] * 5,
        out_specs=pl.BlockSpec(memory_space=pltpu.VMEM),
        scratch_shapes=[
            pltpu.VMEM((2, HALFC, D_MODEL), jnp.float32),
            pltpu.VMEM((2, HALFC, D_MODEL), jnp.float32),
            pltpu.SemaphoreType.DMA((2,)),
            pltpu.SemaphoreType.DMA((2,)),
            pltpu.SemaphoreType.DMA((2,)),
            pltpu.SemaphoreType.DMA((2,)),
        ],
        compiler_params=pltpu.CompilerParams(
            collective_id=0,
            vmem_limit_bytes=100 * 1024 * 1024,
        ),
    )(x2, Wq, K, V, Wo)
    return out.reshape(1, SQ, D_MODEL)


# device time: 113020 ns/iter; 1.8727x vs baseline; 1.2744x over previous
import jax
import jax.numpy as jnp
from jax import lax
from jax.experimental import pallas as pl
from jax.experimental.pallas import tpu as pltpu

N_DEV = 4
SQ = 2048
D_MODEL = 1024
HQ = 8
DH = 128
SCALE = 0.08838834764831843
BLK = 64
N_BLK = SQ // BLK
GROUPS = 4
BLK_PER_GROUP = N_BLK // GROUPS
CHUNK = SQ // N_DEV


HALFC = CHUNK // 2


def _body(x_ref, wq_ref, k_ref, v_ref, wo_ref, out_ref,
          comm_r, comm_l, agb_r, agb_l,
          send_r, recv_r, send_l, recv_l,
          send_ag_r, recv_ag_r, send_ag_l, recv_ag_l):
    my = lax.axis_index("i")
    right = lax.rem(my + 1, N_DEV)
    left = lax.rem(my + N_DEV - 1, N_DEV)
    bf16 = jnp.bfloat16

    wq_b = wq_ref[:, :].astype(bf16)
    wo_b = wo_ref[:, :].astype(bf16)
    for r in range(GROUPS):
        offs = [BLK * (GROUPS * m + r) for m in range(BLK_PER_GROUP)]
        Xr = jnp.concatenate(
            [x_ref[o:o + BLK, :] for o in offs], axis=0).astype(bf16)
        Kr = jnp.concatenate(
            [k_ref[o:o + BLK, :] for o in offs], axis=0).astype(bf16)
        Vr = jnp.concatenate(
            [v_ref[o:o + BLK, :] for o in offs], axis=0).astype(bf16)
        Qr = jnp.dot(Xr, wq_b, preferred_element_type=jnp.float32)
        Qr = Qr.astype(bf16)
        ctx_parts = []
        for h in range(HQ):
            cs = slice(DH * h, DH * (h + 1))
            s = lax.dot_general(
                Qr[:, cs], Kr[:, cs],
                dimension_numbers=(((1,), (1,)), ((), ())),
                preferred_element_type=jnp.float32,
            ) * SCALE
            s = s - jnp.max(s, axis=-1, keepdims=True)
            e = jnp.exp(s)
            w = (e / jnp.sum(e, axis=-1, keepdims=True)).astype(bf16)
            ctx_parts.append(
                jnp.dot(w, Vr[:, cs], preferred_element_type=jnp.float32)
            )
        ctx_r = jnp.concatenate(ctx_parts, axis=1).astype(bf16)
        out_r = jnp.dot(ctx_r, wo_b, preferred_element_type=jnp.float32)
        for m in range(BLK_PER_GROUP):
            out_ref[offs[m]:offs[m] + BLK, :] = out_r[BLK * m:BLK * (m + 1), :]

    barrier_sem = pltpu.get_barrier_semaphore()
    for nbr in (left, right):
        pl.semaphore_signal(
            barrier_sem, inc=1,
            device_id=(nbr,), device_id_type=pl.DeviceIdType.MESH,
        )
    pl.semaphore_wait(barrier_sem, 2)

    for h in range(N_DEV - 1):
        slot = h % 2
        c_sr = lax.rem(my + N_DEV - h, N_DEV)
        c_rr = lax.rem(my + 2 * N_DEV - h - 1, N_DEV)
        c_sl = lax.rem(my + h, N_DEV)
        c_rl = lax.rem(my + h + 1, N_DEV)
        agb_r[slot] = out_ref[pl.ds(c_sr * CHUNK, HALFC), :].astype(bf16)
        agb_l[slot] = out_ref[pl.ds(c_sl * CHUNK + HALFC, HALFC), :].astype(bf16)
        rdma_r = pltpu.make_async_remote_copy(
            src_ref=agb_r.at[slot],
            dst_ref=comm_r.at[slot],
            send_sem=send_r.at[slot],
            recv_sem=recv_r.at[slot],
            device_id=(right,),
            device_id_type=pl.DeviceIdType.MESH,
        )
        rdma_l = pltpu.make_async_remote_copy(
            src_ref=agb_l.at[slot],
            dst_ref=comm_l.at[slot],
            send_sem=send_l.at[slot],
            recv_sem=recv_l.at[slot],
            device_id=(left,),
            device_id_type=pl.DeviceIdType.MESH,
        )
        rdma_r.start()
        rdma_l.start()
        rdma_r.wait()
        dst = pl.ds(c_rr * CHUNK, HALFC)
        out_ref[dst, :] = out_ref[dst, :] + comm_r[slot, :, :].astype(jnp.float32)
        rdma_l.wait()
        dst = pl.ds(c_rl * CHUNK + HALFC, HALFC)
        out_ref[dst, :] = out_ref[dst, :] + comm_l[slot, :, :].astype(jnp.float32)

    own_r = lax.rem(my + 1, N_DEV)
    own_l = lax.rem(my + N_DEV - 1, N_DEV)
    agb_r[0] = out_ref[pl.ds(own_r * CHUNK, HALFC), :].astype(bf16)
    agb_l[0] = out_ref[pl.ds(own_l * CHUNK + HALFC, HALFC), :].astype(bf16)
    for g in range(N_DEV - 1):
        ss = g % 2
        rs = (g + 1) % 2
        c_rr = lax.rem(my + N_DEV - g, N_DEV)
        c_rl = lax.rem(my + g, N_DEV)
        rdma_r = pltpu.make_async_remote_copy(
            src_ref=agb_r.at[ss],
            dst_ref=agb_r.at[rs],
            send_sem=send_ag_r.at[ss],
            recv_sem=recv_ag_r.at[rs],
            device_id=(right,),
            device_id_type=pl.DeviceIdType.MESH,
        )
        rdma_l = pltpu.make_async_remote_copy(
            src_ref=agb_l.at[ss],
            dst_ref=agb_l.at[rs],
            send_sem=send_ag_l.at[ss],
            recv_sem=recv_ag_l.at[rs],
            device_id=(left,),
            device_id_type=pl.DeviceIdType.MESH,
        )
        rdma_r.start()
        rdma_l.start()
        rdma_r.wait()
        out_ref[pl.ds(c_rr * CHUNK, HALFC), :] = agb_r[rs].astype(jnp.float32)
        rdma_l.wait()
        out_ref[pl.ds(c_rl * CHUNK + HALFC, HALFC), :] = agb_l[rs].astype(jnp.float32)


def kernel(x, Wq, K_ext, V_ext, Wo):
    i = lax.axis_index("i")
    x2 = x.reshape(SQ, D_MODEL)
    K = lax.dynamic_slice_in_dim(
        K_ext.reshape(SQ, 32, DH), i * HQ, HQ, axis=1
    ).reshape(SQ, HQ * DH)
    V = lax.dynamic_slice_in_dim(
        V_ext.reshape(SQ, 32, DH), i * HQ, HQ, axis=1
    ).reshape(SQ, HQ * DH)

    out = pl.pallas_call(
        _body,
        out_shape=jax.ShapeDtypeStruct((SQ, D_MODEL), jnp.float32),
        in_specs=[pl.BlockSpec(memory_space=pltpu.VMEM)] * 5,
        out_specs=pl.BlockSpec(memory_space=pltpu.VMEM),
        scratch_shapes=[
            pltpu.VMEM((2, HALFC, D_MODEL), jnp.bfloat16),
            pltpu.VMEM((2, HALFC, D_MODEL), jnp.bfloat16),
            pltpu.VMEM((2, HALFC, D_MODEL), jnp.bfloat16),
            pltpu.VMEM((2, HALFC, D_MODEL), jnp.bfloat16),
            pltpu.SemaphoreType.DMA((2,)),
            pltpu.SemaphoreType.DMA((2,)),
            pltpu.SemaphoreType.DMA((2,)),
            pltpu.SemaphoreType.DMA((2,)),
            pltpu.SemaphoreType.DMA((2,)),
            pltpu.SemaphoreType.DMA((2,)),
            pltpu.SemaphoreType.DMA((2,)),
            pltpu.SemaphoreType.DMA((2,)),
        ],
        compiler_params=pltpu.CompilerParams(
            collective_id=0,
            vmem_limit_bytes=100 * 1024 * 1024,
        ),
    )(x2, Wq, K, V, Wo)
    return out.reshape(1, SQ, D_MODEL)


# device time: 94692 ns/iter; 2.2352x vs baseline; 1.1936x over previous
import jax
import jax.numpy as jnp
from jax import lax
from jax.experimental import pallas as pl
from jax.experimental.pallas import tpu as pltpu

N_DEV = 4
SQ = 2048
D_MODEL = 1024
HQ = 8
DH = 128
SCALE = 0.08838834764831843
BLK = 64
N_BLK = SQ // BLK
GROUPS = 4
BLK_PER_GROUP = N_BLK // GROUPS
CHUNK = SQ // N_DEV


HALFC = CHUNK // 2


def _body(x_ref, wq_ref, kext_ref, vext_ref, wo_ref, out_ref,
          k_vmem, v_vmem,
          comm_r, comm_l, agb_r, agb_l,
          kv_sems,
          send_r, recv_r, send_l, recv_l,
          send_ag_r, recv_ag_r, send_ag_l, recv_ag_l):
    my = lax.axis_index("i")
    right = lax.rem(my + 1, N_DEV)
    left = lax.rem(my + N_DEV - 1, N_DEV)
    bf16 = jnp.bfloat16

    cp_k = pltpu.make_async_copy(
        kext_ref.at[0, :, pl.ds(my * HQ, HQ), :], k_vmem, kv_sems.at[0]
    )
    cp_v = pltpu.make_async_copy(
        vext_ref.at[0, :, pl.ds(my * HQ, HQ), :], v_vmem, kv_sems.at[1]
    )
    cp_k.start()
    cp_v.start()

    wq_b = wq_ref[:, :].astype(bf16)
    wo_b = wo_ref[:, :].astype(bf16)
    cp_k.wait()
    cp_v.wait()
    for r in range(GROUPS):
        offs = [BLK * (GROUPS * m + r) for m in range(BLK_PER_GROUP)]
        Xr = jnp.concatenate(
            [x_ref[o:o + BLK, :] for o in offs], axis=0).astype(bf16)
        Kr = jnp.concatenate(
            [k_vmem[o:o + BLK, :, :] for o in offs], axis=0
        ).reshape(CHUNK, HQ * DH).astype(bf16)
        Vr = jnp.concatenate(
            [v_vmem[o:o + BLK, :, :] for o in offs], axis=0
        ).reshape(CHUNK, HQ * DH).astype(bf16)
        Qr = jnp.dot(Xr, wq_b, preferred_element_type=jnp.float32)
        Qr = Qr.astype(bf16)
        ctx_parts = []
        for h in range(HQ):
            cs = slice(DH * h, DH * (h + 1))
            s = lax.dot_general(
                Qr[:, cs], Kr[:, cs],
                dimension_numbers=(((1,), (1,)), ((), ())),
                preferred_element_type=jnp.float32,
            ) * SCALE
            s = s - jnp.max(s, axis=-1, keepdims=True)
            e = jnp.exp(s)
            w = (e / jnp.sum(e, axis=-1, keepdims=True)).astype(bf16)
            ctx_parts.append(
                jnp.dot(w, Vr[:, cs], preferred_element_type=jnp.float32)
            )
        ctx_r = jnp.concatenate(ctx_parts, axis=1).astype(bf16)
        out_r = jnp.dot(ctx_r, wo_b, preferred_element_type=jnp.float32)
        for m in range(BLK_PER_GROUP):
            out_ref[offs[m]:offs[m] + BLK, :] = out_r[BLK * m:BLK * (m + 1), :]

    barrier_sem = pltpu.get_barrier_semaphore()
    for nbr in (left, right):
        pl.semaphore_signal(
            barrier_sem, inc=1,
            device_id=(nbr,), device_id_type=pl.DeviceIdType.MESH,
        )
    pl.semaphore_wait(barrier_sem, 2)

    for h in range(N_DEV - 1):
        slot = h % 2
        c_sr = lax.rem(my + N_DEV - h, N_DEV)
        c_rr = lax.rem(my + 2 * N_DEV - h - 1, N_DEV)
        c_sl = lax.rem(my + h, N_DEV)
        c_rl = lax.rem(my + h + 1, N_DEV)
        agb_r[slot] = out_ref[pl.ds(c_sr * CHUNK, HALFC), :].astype(bf16)
        agb_l[slot] = out_ref[pl.ds(c_sl * CHUNK + HALFC, HALFC), :].astype(bf16)
        rdma_r = pltpu.make_async_remote_copy(
            src_ref=agb_r.at[slot],
            dst_ref=comm_r.at[slot],
            send_sem=send_r.at[slot],
            recv_sem=recv_r.at[slot],
            device_id=(right,),
            device_id_type=pl.DeviceIdType.MESH,
        )
        rdma_l = pltpu.make_async_remote_copy(
            src_ref=agb_l.at[slot],
            dst_ref=comm_l.at[slot],
            send_sem=send_l.at[slot],
            recv_sem=recv_l.at[slot],
            device_id=(left,),
            device_id_type=pl.DeviceIdType.MESH,
        )
        rdma_r.start()
        rdma_l.start()
        rdma_r.wait()
        dst = pl.ds(c_rr * CHUNK, HALFC)
        out_ref[dst, :] = out_ref[dst, :] + comm_r[slot, :, :].astype(jnp.float32)
        rdma_l.wait()
        dst = pl.ds(c_rl * CHUNK + HALFC, HALFC)
        out_ref[dst, :] = out_ref[dst, :] + comm_l[slot, :, :].astype(jnp.float32)

    own_r = lax.rem(my + 1, N_DEV)
    own_l = lax.rem(my + N_DEV - 1, N_DEV)
    agb_r[0] = out_ref[pl.ds(own_r * CHUNK, HALFC), :].astype(bf16)
    agb_l[0] = out_ref[pl.ds(own_l * CHUNK + HALFC, HALFC), :].astype(bf16)
    for g in range(N_DEV - 1):
        ss = g % 2
        rs = (g + 1) % 2
        c_rr = lax.rem(my + N_DEV - g, N_DEV)
        c_rl = lax.rem(my + g, N_DEV)
        rdma_r = pltpu.make_async_remote_copy(
            src_ref=agb_r.at[ss],
            dst_ref=agb_r.at[rs],
            send_sem=send_ag_r.at[ss],
            recv_sem=recv_ag_r.at[rs],
            device_id=(right,),
            device_id_type=pl.DeviceIdType.MESH,
        )
        rdma_l = pltpu.make_async_remote_copy(
            src_ref=agb_l.at[ss],
            dst_ref=agb_l.at[rs],
            send_sem=send_ag_l.at[ss],
            recv_sem=recv_ag_l.at[rs],
            device_id=(left,),
            device_id_type=pl.DeviceIdType.MESH,
        )
        rdma_r.start()
        rdma_l.start()
        rdma_r.wait()
        out_ref[pl.ds(c_rr * CHUNK, HALFC), :] = agb_r[rs].astype(jnp.float32)
        rdma_l.wait()
        out_ref[pl.ds(c_rl * CHUNK + HALFC, HALFC), :] = agb_l[rs].astype(jnp.float32)


def kernel(x, Wq, K_ext, V_ext, Wo):
    x2 = x.reshape(SQ, D_MODEL)

    out = pl.pallas_call(
        _body,
        out_shape=jax.ShapeDtypeStruct((SQ, D_MODEL), jnp.float32),
        in_specs=[
            pl.BlockSpec(memory_space=pltpu.VMEM),
            pl.BlockSpec(memory_space=pltpu.VMEM),
            pl.BlockSpec(memory_space=pltpu.MemorySpace.HBM),
            pl.BlockSpec(memory_space=pltpu.MemorySpace.HBM),
            pl.BlockSpec(memory_space=pltpu.VMEM),
        ],
        out_specs=pl.BlockSpec(memory_space=pltpu.VMEM),
        scratch_shapes=[
            pltpu.VMEM((SQ, HQ, DH), jnp.float32),
            pltpu.VMEM((SQ, HQ, DH), jnp.float32),
            pltpu.VMEM((2, HALFC, D_MODEL), jnp.bfloat16),
            pltpu.VMEM((2, HALFC, D_MODEL), jnp.bfloat16),
            pltpu.VMEM((2, HALFC, D_MODEL), jnp.bfloat16),
            pltpu.VMEM((2, HALFC, D_MODEL), jnp.bfloat16),
            pltpu.SemaphoreType.DMA((2,)),
            pltpu.SemaphoreType.DMA((2,)),
            pltpu.SemaphoreType.DMA((2,)),
            pltpu.SemaphoreType.DMA((2,)),
            pltpu.SemaphoreType.DMA((2,)),
            pltpu.SemaphoreType.DMA((2,)),
            pltpu.SemaphoreType.DMA((2,)),
            pltpu.SemaphoreType.DMA((2,)),
            pltpu.SemaphoreType.DMA((2,)),
        ],
        compiler_params=pltpu.CompilerParams(
            collective_id=0,
            vmem_limit_bytes=100 * 1024 * 1024,
        ),
    )(x2, Wq, K_ext, V_ext, Wo)
    return out.reshape(1, SQ, D_MODEL)


# device time: 81937 ns/iter; 2.5831x vs baseline; 1.1557x over previous
import jax
import jax.numpy as jnp
from jax import lax
from jax.experimental import pallas as pl
from jax.experimental.pallas import tpu as pltpu

N_DEV = 4
SQ = 2048
D_MODEL = 1024
HQ = 8
DH = 128
SCALE = 0.08838834764831843
BLK = 64
N_BLK = SQ // BLK
GROUPS = 4
BLK_PER_GROUP = N_BLK // GROUPS
CHUNK = SQ // N_DEV
HALFC = CHUNK // 2
GRP_ROWS = BLK * BLK_PER_GROUP
GC = GRP_ROWS // N_DEV
N_HOP = 2 * (N_DEV - 1)
N_TICK = N_DEV + N_HOP


def _body(x_ref, wq_ref, kext_ref, vext_ref, wo_ref, out_ref,
          k_vmem, v_vmem, grp_ref,
          cgr, cgl, agr, agl,
          kv_sems,
          rs_send_r, rs_recv_r, rs_send_l, rs_recv_l,
          ag_send_r, ag_recv_r, ag_send_l, ag_recv_l):
    my = lax.axis_index("i")
    right = lax.rem(my + 1, N_DEV)
    left = lax.rem(my + N_DEV - 1, N_DEV)
    own_r = right
    own_l = left
    bf16 = jnp.bfloat16
    f32 = jnp.float32

    cp_k = pltpu.make_async_copy(
        kext_ref.at[0, :, pl.ds(my * HQ, HQ), :], k_vmem, kv_sems.at[0]
    )
    cp_v = pltpu.make_async_copy(
        vext_ref.at[0, :, pl.ds(my * HQ, HQ), :], v_vmem, kv_sems.at[1]
    )
    cp_k.start()
    cp_v.start()

    wq_b = wq_ref[:, :].astype(bf16)
    wo_b = wo_ref[:, :].astype(bf16)
    cp_k.wait()
    cp_v.wait()

    def compute_group(r):
        offs = [BLK * (GROUPS * m + r) for m in range(BLK_PER_GROUP)]
        Xr = jnp.concatenate(
            [x_ref[o:o + BLK, :] for o in offs], axis=0).astype(bf16)
        Kr = jnp.concatenate(
            [k_vmem[o:o + BLK, :, :] for o in offs], axis=0
        ).reshape(GRP_ROWS, HQ * DH).astype(bf16)
        Vr = jnp.concatenate(
            [v_vmem[o:o + BLK, :, :] for o in offs], axis=0
        ).reshape(GRP_ROWS, HQ * DH).astype(bf16)
        Qr = jnp.dot(Xr, wq_b, preferred_element_type=f32).astype(bf16)
        ctx_parts = []
        for h in range(HQ):
            cs = slice(DH * h, DH * (h + 1))
            s = lax.dot_general(
                Qr[:, cs], Kr[:, cs],
                dimension_numbers=(((1,), (1,)), ((), ())),
                preferred_element_type=f32,
            ) * SCALE
            s = s - jnp.max(s, axis=-1, keepdims=True)
            e = jnp.exp(s)
            w = (e / jnp.sum(e, axis=-1, keepdims=True)).astype(bf16)
            ctx_parts.append(
                jnp.dot(w, Vr[:, cs], preferred_element_type=f32)
            )
        ctx_r = jnp.concatenate(ctx_parts, axis=1).astype(bf16)
        grp_ref[r, :, :] = jnp.dot(
            ctx_r, wo_b, preferred_element_type=f32).astype(bf16)

    descs = {}

    def issue_hop(r, s):
        if s < N_DEV - 1:
            h = s
            c_sr = lax.rem(my + N_DEV - h, N_DEV)
            c_sl = lax.rem(my + h, N_DEV)
            d_r = pltpu.make_async_remote_copy(
                src_ref=grp_ref.at[r, pl.ds(c_sr * GC, BLK), :],
                dst_ref=cgr.at[r, h],
                send_sem=rs_send_r.at[r, h], recv_sem=rs_recv_r.at[r, h],
                device_id=(right,), device_id_type=pl.DeviceIdType.MESH,
            )
            d_l = pltpu.make_async_remote_copy(
                src_ref=grp_ref.at[r, pl.ds(c_sl * GC + BLK, BLK), :],
                dst_ref=cgl.at[r, h],
                send_sem=rs_send_l.at[r, h], recv_sem=rs_recv_l.at[r, h],
                device_id=(left,), device_id_type=pl.DeviceIdType.MESH,
            )
            d_r.start()
            d_l.start()
        else:
            g = s - (N_DEV - 1)
            src_r = (grp_ref.at[r, pl.ds(own_r * GC, BLK), :] if g == 0
                     else agr.at[r, g - 1])
            src_l = (grp_ref.at[r, pl.ds(own_l * GC + BLK, BLK), :] if g == 0
                     else agl.at[r, g - 1])
            d_r = pltpu.make_async_remote_copy(
                src_ref=src_r, dst_ref=agr.at[r, g],
                send_sem=ag_send_r.at[r, g], recv_sem=ag_recv_r.at[r, g],
                device_id=(right,), device_id_type=pl.DeviceIdType.MESH,
            )
            d_l = pltpu.make_async_remote_copy(
                src_ref=src_l, dst_ref=agl.at[r, g],
                send_sem=ag_send_l.at[r, g], recv_sem=ag_recv_l.at[r, g],
                device_id=(left,), device_id_type=pl.DeviceIdType.MESH,
            )
            d_r.start()
            d_l.start()
        descs[(r, s)] = (d_r, d_l)

    def wait_hop(r, s):
        d_r, d_l = descs[(r, s)]
        if s < N_DEV - 1:
            h = s
            c_rr = lax.rem(my + 2 * N_DEV - h - 1, N_DEV)
            c_rl = lax.rem(my + h + 1, N_DEV)
            d_r.wait()
            dst = pl.ds(c_rr * GC, BLK)
            grp_ref[r, dst, :] = (grp_ref[r, dst, :].astype(f32)
                                  + cgr[r, h].astype(f32)).astype(bf16)
            d_l.wait()
            dst = pl.ds(c_rl * GC + BLK, BLK)
            grp_ref[r, dst, :] = (grp_ref[r, dst, :].astype(f32)
                                  + cgl[r, h].astype(f32)).astype(bf16)
            if h == N_DEV - 2:
                out_ref[pl.ds(own_r * CHUNK + r * BLK, BLK), :] = (
                    grp_ref[r, pl.ds(own_r * GC, BLK), :].astype(f32))
                out_ref[pl.ds(own_l * CHUNK + HALFC + r * BLK, BLK), :] = (
                    grp_ref[r, pl.ds(own_l * GC + BLK, BLK), :].astype(f32))
        else:
            g = s - (N_DEV - 1)
            c_rr = lax.rem(my + N_DEV - g, N_DEV)
            c_rl = lax.rem(my + g, N_DEV)
            d_r.wait()
            out_ref[pl.ds(c_rr * CHUNK + r * BLK, BLK), :] = (
                agr[r, g].astype(f32))
            d_l.wait()
            out_ref[pl.ds(c_rl * CHUNK + HALFC + r * BLK, BLK), :] = (
                agl[r, g].astype(f32))

    for tick in range(N_TICK):
        if tick < GROUPS:
            compute_group(tick)
        if tick == 0:
            barrier_sem = pltpu.get_barrier_semaphore()
            for nbr in (left, right):
                pl.semaphore_signal(
                    barrier_sem, inc=1,
                    device_id=(nbr,), device_id_type=pl.DeviceIdType.MESH,
                )
            pl.semaphore_wait(barrier_sem, 2)
        for r in range(GROUPS):
            s = tick - r - 1
            if 0 <= s <= N_HOP - 1:
                wait_hop(r, s)
                if s < N_HOP - 1:
                    issue_hop(r, s + 1)
        if tick < GROUPS:
            issue_hop(tick, 0)


def kernel(x, Wq, K_ext, V_ext, Wo):
    x2 = x.reshape(SQ, D_MODEL)

    out = pl.pallas_call(
        _body,
        out_shape=jax.ShapeDtypeStruct((SQ, D_MODEL), jnp.float32),
        in_specs=[
            pl.BlockSpec(memory_space=pltpu.MemorySpace.VMEM),
            pl.BlockSpec(memory_space=pltpu.MemorySpace.VMEM),
            pl.BlockSpec(memory_space=pltpu.MemorySpace.HBM),
            pl.BlockSpec(memory_space=pltpu.MemorySpace.HBM),
            pl.BlockSpec(memory_space=pltpu.MemorySpace.VMEM),
        ],
        out_specs=pl.BlockSpec(memory_space=pltpu.MemorySpace.VMEM),
        scratch_shapes=[
            pltpu.VMEM((SQ, HQ, DH), jnp.float32),
            pltpu.VMEM((SQ, HQ, DH), jnp.float32),
            pltpu.VMEM((GROUPS, GRP_ROWS, D_MODEL), jnp.bfloat16),
            pltpu.VMEM((GROUPS, 3, BLK, D_MODEL), jnp.bfloat16),
            pltpu.VMEM((GROUPS, 3, BLK, D_MODEL), jnp.bfloat16),
            pltpu.VMEM((GROUPS, 3, BLK, D_MODEL), jnp.bfloat16),
            pltpu.VMEM((GROUPS, 3, BLK, D_MODEL), jnp.bfloat16),
            pltpu.SemaphoreType.DMA((2,)),
            pltpu.SemaphoreType.DMA((GROUPS, 3)),
            pltpu.SemaphoreType.DMA((GROUPS, 3)),
            pltpu.SemaphoreType.DMA((GROUPS, 3)),
            pltpu.SemaphoreType.DMA((GROUPS, 3)),
            pltpu.SemaphoreType.DMA((GROUPS, 3)),
            pltpu.SemaphoreType.DMA((GROUPS, 3)),
            pltpu.SemaphoreType.DMA((GROUPS, 3)),
            pltpu.SemaphoreType.DMA((GROUPS, 3)),
        ],
        compiler_params=pltpu.CompilerParams(
            collective_id=0,
            vmem_limit_bytes=100 * 1024 * 1024,
        ),
    )(x2, Wq, K_ext, V_ext, Wo)
    return out.reshape(1, SQ, D_MODEL)
